# Initial kernel scaffold; baseline (speedup 1.0000x reference)
import jax
import jax.numpy as jnp
from jax.experimental import pallas as pl


def kernel(x, edge_index, batch, W1, b1, W2, b2, W3, b3):
    raise NotImplementedError("write your pallas kernel here")


# speedup vs baseline: 2.5651x; 2.5651x over previous
import functools
import jax
import jax.numpy as jnp
from jax.experimental import pallas as pl
from jax.experimental.pallas import tpu as pltpu

_N = 50000
_E = 800000
_G = 1024
_H = 128
_BLK = 512
_NPAD = 50176


def _mm_first(x_ref, w_ref, dinv_ref, g_ref):
    g_ref[...] = jnp.dot(x_ref[...], w_ref[...],
                         preferred_element_type=jnp.float32) * dinv_ref[...]


def _mm_stage(s_ref, g_ref, dinv_ref, b_ref, w_ref, out_ref):
    h = jnp.maximum(dinv_ref[...] * (s_ref[...] + g_ref[...]) + b_ref[...], 0.0)
    out_ref[...] = jnp.dot(h, w_ref[...],
                           preferred_element_type=jnp.float32) * dinv_ref[...]


def _mm_last(s_ref, g_ref, dinv_ref, b_ref, out_ref):
    out_ref[...] = jnp.maximum(
        dinv_ref[...] * (s_ref[...] + g_ref[...]) + b_ref[...], 0.0)


_row_spec = pl.BlockSpec((_BLK, _H), lambda i: (i, 0))
_dinv_spec = pl.BlockSpec((_BLK, 1), lambda i: (i, 0))
_w_spec = pl.BlockSpec((_H, _H), lambda i: (0, 0))
_b_spec = pl.BlockSpec((1, _H), lambda i: (0, 0))
_grid = (_NPAD // _BLK,)
_row_out = jax.ShapeDtypeStruct((_NPAD, _H), jnp.float32)


def _first_layer(x_pad, w1_pad, dinv):
    return pl.pallas_call(
        _mm_first, grid=_grid,
        in_specs=[_row_spec, _w_spec, _dinv_spec],
        out_specs=_row_spec, out_shape=_row_out,
    )(x_pad, w1_pad, dinv)


def _mid_layer(s, g, dinv, b, w):
    return pl.pallas_call(
        _mm_stage, grid=_grid,
        in_specs=[_row_spec, _row_spec, _dinv_spec, _b_spec, _w_spec],
        out_specs=_row_spec, out_shape=_row_out,
    )(s, g, dinv, b, w)


def _last_layer(s, g, dinv, b):
    return pl.pallas_call(
        _mm_last, grid=_grid,
        in_specs=[_row_spec, _row_spec, _dinv_spec, _b_spec],
        out_specs=_row_spec, out_shape=_row_out,
    )(s, g, dinv, b)


def kernel(x, edge_index, batch, W1, b1, W2, b2, W3, b3):
    src = edge_index[0]
    dst = edge_index[1]
    indeg = jax.ops.segment_sum(jnp.ones((_E,), jnp.float32), dst,
                                num_segments=_N)
    dinv = jax.lax.rsqrt(indeg + 1.0)
    dinv = jnp.pad(dinv, (0, _NPAD - _N)).reshape(_NPAD, 1)

    x_pad = jnp.pad(x, ((0, _NPAD - _N), (0, _H - x.shape[1])))
    w1_pad = jnp.pad(W1, ((0, _H - W1.shape[0]), (0, 0)))

    def scat(g):
        return jnp.pad(
            jax.ops.segment_sum(g[:_N][src], dst, num_segments=_N),
            ((0, _NPAD - _N), (0, 0)))

    g1 = _first_layer(x_pad, w1_pad, dinv)
    s1 = scat(g1)
    g2 = _mid_layer(s1, g1, dinv, b1.reshape(1, _H), W2)
    s2 = scat(g2)
    g3 = _mid_layer(s2, g2, dinv, b2.reshape(1, _H), W3)
    s3 = scat(g3)
    h3 = _last_layer(s3, g3, dinv, b3.reshape(1, _H))
    return jax.ops.segment_max(h3[:_N], batch, num_segments=_G)


# speedup vs baseline: 5.9186x; 2.3073x over previous
import functools
import jax
import jax.numpy as jnp
from jax import lax
from jax.experimental import pallas as pl
from jax.experimental.pallas import tpu as pltpu
from jax.experimental.pallas import tpu_sc as plsc

_N = 50000
_E = 800000
_G = 1024
_H = 128
_BLK = 512
_NPAD = 50176

_C = 32
_RPS = _NPAD // 16
_EPAD = 819200
_EROWS = _EPAD // 128
_ERPS = _EROWS // 16
_NB = 16
_NOUT = _ERPS // _NB


def _mm_first(x_ref, w_ref, dinv_ref, g_ref):
    g_ref[...] = jnp.dot(x_ref[...], w_ref[...],
                         preferred_element_type=jnp.float32) * dinv_ref[...]


def _mm_stage(s_ref, g_ref, dinv_ref, b_ref, w_ref, out_ref):
    h = jnp.maximum(dinv_ref[...] * (s_ref[...] + g_ref[...]) + b_ref[...], 0.0)
    out_ref[...] = jnp.dot(h, w_ref[...],
                           preferred_element_type=jnp.float32) * dinv_ref[...]


def _mm_last(s_ref, g_ref, dinv_ref, b_ref, out_ref):
    out_ref[...] = jnp.maximum(
        dinv_ref[...] * (s_ref[...] + g_ref[...]) + b_ref[...], 0.0)


_row_spec = pl.BlockSpec((_BLK, _H), lambda i: (i, 0))
_dinv_spec = pl.BlockSpec((_BLK, 1), lambda i: (i, 0))
_w_spec = pl.BlockSpec((_H, _H), lambda i: (0, 0))
_b_spec = pl.BlockSpec((1, _H), lambda i: (0, 0))
_grid = (_NPAD // _BLK,)
_row_out = jax.ShapeDtypeStruct((_NPAD, _H), jnp.float32)


def _first_layer(x_pad, w1_pad, dinv):
    return pl.pallas_call(
        _mm_first, grid=_grid,
        in_specs=[_row_spec, _w_spec, _dinv_spec],
        out_specs=_row_spec, out_shape=_row_out,
    )(x_pad, w1_pad, dinv)


def _mid_layer(s, g, dinv, b, w):
    return pl.pallas_call(
        _mm_stage, grid=_grid,
        in_specs=[_row_spec, _row_spec, _dinv_spec, _b_spec, _w_spec],
        out_specs=_row_spec, out_shape=_row_out,
    )(s, g, dinv, b, w)


def _last_layer(s, g, dinv, b):
    return pl.pallas_call(
        _mm_last, grid=_grid,
        in_specs=[_row_spec, _row_spec, _dinv_spec, _b_spec],
        out_specs=_row_spec, out_shape=_row_out,
    )(s, g, dinv, b)


def _sc_body(zeros, srcr, dstr, g0, g1, g2, g3, o0, o1, o2, o3,
             src_idx, dst_idx, rows, sem0, sem1, acc):
    c = lax.axis_index("c")
    s = lax.axis_index("s")
    row0 = s * _RPS

    def run_chunk(g_ref, out_ref):
        pltpu.sync_copy(zeros.at[pl.ds(row0, _RPS)], acc.at[pl.ds(row0, _RPS)])
        plsc.subcore_barrier()
        ebase = s * _ERPS

        def outer(i, carry):
            base = ebase + i * _NB
            pltpu.sync_copy(srcr.at[pl.ds(base, _NB)], src_idx)
            pltpu.sync_copy(dstr.at[pl.ds(base, _NB)], dst_idx)
            cps = [None, None]
            cps[0] = pltpu.async_copy(g_ref.at[src_idx.at[0]], rows.at[0], sem0)
            for j in range(_NB):
                b = j % 2
                if j + 1 < _NB:
                    nb = (j + 1) % 2
                    cps[nb] = pltpu.async_copy(
                        g_ref.at[src_idx.at[j + 1]], rows.at[nb],
                        sem1 if nb else sem0)
                cps[b].wait()
                pltpu.sync_copy(rows.at[b], acc.at[dst_idx.at[j]], add=True)
            return carry

        lax.fori_loop(0, _NOUT, outer, 0)
        plsc.subcore_barrier()
        pltpu.sync_copy(acc.at[pl.ds(row0, _RPS)], out_ref.at[pl.ds(row0, _RPS)])

    @pl.when(c == 0)
    def _():
        run_chunk(g0, o0)
        run_chunk(g1, o1)

    @pl.when(c == 1)
    def _():
        run_chunk(g2, o2)
        run_chunk(g3, o3)


_sc_scatter = pl.kernel(
    _sc_body,
    out_type=[jax.ShapeDtypeStruct((_NPAD, _C), jnp.float32)] * 4,
    mesh=plsc.VectorSubcoreMesh(core_axis_name="c", subcore_axis_name="s"),
    compiler_params=pltpu.CompilerParams(use_tc_tiling_on_sc=False),
    scratch_types=[
        pltpu.VMEM((_NB, 128), jnp.int32),
        pltpu.VMEM((_NB, 128), jnp.int32),
        pltpu.VMEM((2, 128, _C), jnp.float32),
        pltpu.SemaphoreType.DMA,
        pltpu.SemaphoreType.DMA,
        pltpu.VMEM_SHARED((_NPAD, _C), jnp.float32),
    ],
)


def kernel(x, edge_index, batch, W1, b1, W2, b2, W3, b3):
    src = edge_index[0]
    dst = edge_index[1]
    indeg = jax.ops.segment_sum(jnp.ones((_E,), jnp.float32), dst,
                                num_segments=_N)
    dinv = jax.lax.rsqrt(indeg + 1.0)
    dinv = jnp.pad(dinv, (0, _NPAD - _N)).reshape(_NPAD, 1)

    x_pad = jnp.pad(x, ((0, _NPAD - _N), (0, _H - x.shape[1])))
    w1_pad = jnp.pad(W1, ((0, _H - W1.shape[0]), (0, 0)))

    pad_i = jnp.full((_EPAD - _E,), _NPAD - 1, jnp.int32)
    srcp = jnp.concatenate([src, pad_i]).reshape(_EROWS, 128)
    dstp = jnp.concatenate([dst, pad_i]).reshape(_EROWS, 128)
    zeros = jnp.zeros((_NPAD, _C), jnp.float32)

    def scat(g):
        chunks = _sc_scatter(zeros, srcp, dstp,
                             g[:, 0:32], g[:, 32:64], g[:, 64:96], g[:, 96:128])
        return jnp.concatenate(chunks, axis=1)

    g1 = _first_layer(x_pad, w1_pad, dinv)
    s1 = scat(g1)
    g2 = _mid_layer(s1, g1, dinv, b1.reshape(1, _H), W2)
    s2 = scat(g2)
    g3 = _mid_layer(s2, g2, dinv, b2.reshape(1, _H), W3)
    s3 = scat(g3)
    h3 = _last_layer(s3, g3, dinv, b3.reshape(1, _H))
    return jax.ops.segment_max(h3[:_N], batch, num_segments=_G)


# speedup vs baseline: 6.2135x; 1.0498x over previous
import functools
import jax
import jax.numpy as jnp
from jax import lax
from jax.experimental import pallas as pl
from jax.experimental.pallas import tpu as pltpu
from jax.experimental.pallas import tpu_sc as plsc

_N = 50000
_E = 800000
_G = 1024
_H = 128
_BLK = 512
_NPAD = 50176

_C = 32
_RPS = _NPAD // 16
_EPAD = 819200
_EROWS = _EPAD // 128
_ERPS = _EROWS // 16
_NB = 16
_NOUT = _ERPS // _NB


def _mm_first(x_ref, w_ref, dinv_ref, g_ref):
    g_ref[...] = jnp.dot(x_ref[...], w_ref[...],
                         preferred_element_type=jnp.float32) * dinv_ref[...]


def _mm_stage(s_ref, g_ref, dinv_ref, b_ref, w_ref, out_ref):
    h = jnp.maximum(dinv_ref[...] * (s_ref[...] + g_ref[...]) + b_ref[...], 0.0)
    out_ref[...] = jnp.dot(h, w_ref[...],
                           preferred_element_type=jnp.float32) * dinv_ref[...]


def _mm_last(s_ref, g_ref, dinv_ref, b_ref, out_ref):
    out_ref[...] = jnp.maximum(
        dinv_ref[...] * (s_ref[...] + g_ref[...]) + b_ref[...], 0.0)


_row_spec = pl.BlockSpec((_BLK, _H), lambda i: (i, 0))
_dinv_spec = pl.BlockSpec((_BLK, 1), lambda i: (i, 0))
_w_spec = pl.BlockSpec((_H, _H), lambda i: (0, 0))
_b_spec = pl.BlockSpec((1, _H), lambda i: (0, 0))
_grid = (_NPAD // _BLK,)
_row_out = jax.ShapeDtypeStruct((_NPAD, _H), jnp.float32)


def _first_layer(x_pad, w1_pad, dinv):
    return pl.pallas_call(
        _mm_first, grid=_grid,
        in_specs=[_row_spec, _w_spec, _dinv_spec],
        out_specs=_row_spec, out_shape=_row_out,
    )(x_pad, w1_pad, dinv)


def _mid_layer(s, g, dinv, b, w):
    return pl.pallas_call(
        _mm_stage, grid=_grid,
        in_specs=[_row_spec, _row_spec, _dinv_spec, _b_spec, _w_spec],
        out_specs=_row_spec, out_shape=_row_out,
    )(s, g, dinv, b, w)


def _last_layer(s, g, dinv, b):
    return pl.pallas_call(
        _mm_last, grid=_grid,
        in_specs=[_row_spec, _row_spec, _dinv_spec, _b_spec],
        out_specs=_row_spec, out_shape=_row_out,
    )(s, g, dinv, b)


_NBUF = 4


def _sc_body(zeros, srcr, dstr, g0, g1, g2, g3, o0, o1, o2, o3,
             src_idx, dst_idx, rows, gsems, ssems, acc):
    c = lax.axis_index("c")
    s = lax.axis_index("s")
    row0 = s * _RPS

    def run_chunk(g_ref, out_ref):
        pltpu.sync_copy(zeros.at[pl.ds(row0, _RPS)], acc.at[pl.ds(row0, _RPS)])
        plsc.subcore_barrier()
        ebase = s * _ERPS

        def outer(i, carry):
            base = ebase + i * _NB
            pltpu.sync_copy(srcr.at[pl.ds(base, _NB)], src_idx)
            pltpu.sync_copy(dstr.at[pl.ds(base, _NB)], dst_idx)
            gcp = [None] * _NB
            scp = [None] * _NB

            def gather(j):
                b = j % _NBUF
                gcp[j] = pltpu.async_copy(
                    g_ref.at[src_idx.at[j]], rows.at[b], gsems.at[b])

            for j in range(_NBUF - 1):
                gather(j)
            for j in range(_NB):
                b = j % _NBUF
                gcp[j].wait()
                scp[j] = pltpu.async_copy(
                    rows.at[b], acc.at[dst_idx.at[j]], ssems.at[b], add=True)
                nxt = j + _NBUF - 1
                if nxt < _NB:
                    if nxt >= _NBUF:
                        scp[nxt - _NBUF].wait()
                    gather(nxt)
            for j in range(_NB - _NBUF, _NB):
                scp[j].wait()
            return carry

        lax.fori_loop(0, _NOUT, outer, 0)
        plsc.subcore_barrier()
        pltpu.sync_copy(acc.at[pl.ds(row0, _RPS)], out_ref.at[pl.ds(row0, _RPS)])

    @pl.when(c == 0)
    def _():
        run_chunk(g0, o0)
        run_chunk(g1, o1)

    @pl.when(c == 1)
    def _():
        run_chunk(g2, o2)
        run_chunk(g3, o3)


_sc_scatter = pl.kernel(
    _sc_body,
    out_type=[jax.ShapeDtypeStruct((_NPAD, _C), jnp.float32)] * 4,
    mesh=plsc.VectorSubcoreMesh(core_axis_name="c", subcore_axis_name="s"),
    compiler_params=pltpu.CompilerParams(use_tc_tiling_on_sc=False),
    scratch_types=[
        pltpu.VMEM((_NB, 128), jnp.int32),
        pltpu.VMEM((_NB, 128), jnp.int32),
        pltpu.VMEM((_NBUF, 128, _C), jnp.float32),
        pltpu.SemaphoreType.DMA((_NBUF,)),
        pltpu.SemaphoreType.DMA((_NBUF,)),
        pltpu.VMEM_SHARED((_NPAD, _C), jnp.float32),
    ],
)


def kernel(x, edge_index, batch, W1, b1, W2, b2, W3, b3):
    src = edge_index[0]
    dst = edge_index[1]
    indeg = jax.ops.segment_sum(jnp.ones((_E,), jnp.float32), dst,
                                num_segments=_N)
    dinv = jax.lax.rsqrt(indeg + 1.0)
    dinv = jnp.pad(dinv, (0, _NPAD - _N)).reshape(_NPAD, 1)

    x_pad = jnp.pad(x, ((0, _NPAD - _N), (0, _H - x.shape[1])))
    w1_pad = jnp.pad(W1, ((0, _H - W1.shape[0]), (0, 0)))

    pad_i = jnp.full((_EPAD - _E,), _NPAD - 1, jnp.int32)
    srcp = jnp.concatenate([src, pad_i]).reshape(_EROWS, 128)
    dstp = jnp.concatenate([dst, pad_i]).reshape(_EROWS, 128)
    zeros = jnp.zeros((_NPAD, _C), jnp.float32)

    def scat(g):
        chunks = _sc_scatter(zeros, srcp, dstp,
                             g[:, 0:32], g[:, 32:64], g[:, 64:96], g[:, 96:128])
        return jnp.concatenate(chunks, axis=1)

    g1 = _first_layer(x_pad, w1_pad, dinv)
    s1 = scat(g1)
    g2 = _mid_layer(s1, g1, dinv, b1.reshape(1, _H), W2)
    s2 = scat(g2)
    g3 = _mid_layer(s2, g2, dinv, b2.reshape(1, _H), W3)
    s3 = scat(g3)
    h3 = _last_layer(s3, g3, dinv, b3.reshape(1, _H))
    return jax.ops.segment_max(h3[:_N], batch, num_segments=_G)


# speedup vs baseline: 6.4315x; 1.0351x over previous
import functools
import jax
import jax.numpy as jnp
from jax import lax
from jax.experimental import pallas as pl
from jax.experimental.pallas import tpu as pltpu
from jax.experimental.pallas import tpu_sc as plsc

_N = 50000
_E = 800000
_G = 1024
_H = 128
_BLK = 512
_NPAD = 50176

_C = 32
_RPS = _NPAD // 16
_EPAD = 819200
_EROWS = _EPAD // 128
_ERPS = _EROWS // 16
_NB = 16
_NOUT = _ERPS // _NB


def _mm_first(x_ref, w_ref, dinv_ref, o0, o1, o2, o3):
    g = jnp.dot(x_ref[...], w_ref[...],
                preferred_element_type=jnp.float32) * dinv_ref[...]
    for k, o in enumerate((o0, o1, o2, o3)):
        o[...] = g[:, _C * k:_C * (k + 1)]


def _mm_stage(s0, s1, s2, s3, g0, g1, g2, g3, dinv_ref, b_ref, w_ref,
              o0, o1, o2, o3):
    s = jnp.concatenate([s0[...], s1[...], s2[...], s3[...]], axis=1)
    g = jnp.concatenate([g0[...], g1[...], g2[...], g3[...]], axis=1)
    h = jnp.maximum(dinv_ref[...] * (s + g) + b_ref[...], 0.0)
    out = jnp.dot(h, w_ref[...],
                  preferred_element_type=jnp.float32) * dinv_ref[...]
    for k, o in enumerate((o0, o1, o2, o3)):
        o[...] = out[:, _C * k:_C * (k + 1)]


def _mm_last(s0, s1, s2, s3, g0, g1, g2, g3, dinv_ref, b_ref, out_ref):
    s = jnp.concatenate([s0[...], s1[...], s2[...], s3[...]], axis=1)
    g = jnp.concatenate([g0[...], g1[...], g2[...], g3[...]], axis=1)
    out_ref[...] = jnp.maximum(dinv_ref[...] * (s + g) + b_ref[...], 0.0)


_row_spec = pl.BlockSpec((_BLK, _H), lambda i: (i, 0))
_chunk_spec = pl.BlockSpec((_BLK, _C), lambda i: (i, 0))
_dinv_spec = pl.BlockSpec((_BLK, 1), lambda i: (i, 0))
_w_spec = pl.BlockSpec((_H, _H), lambda i: (0, 0))
_b_spec = pl.BlockSpec((1, _H), lambda i: (0, 0))
_grid = (_NPAD // _BLK,)
_row_out = jax.ShapeDtypeStruct((_NPAD, _H), jnp.float32)
_chunk_out = [jax.ShapeDtypeStruct((_NPAD, _C), jnp.float32)] * 4


def _first_layer(x_pad, w1_pad, dinv):
    return pl.pallas_call(
        _mm_first, grid=_grid,
        in_specs=[_row_spec, _w_spec, _dinv_spec],
        out_specs=[_chunk_spec] * 4, out_shape=_chunk_out,
    )(x_pad, w1_pad, dinv)


def _mid_layer(sc, gc, dinv, b, w):
    return pl.pallas_call(
        _mm_stage, grid=_grid,
        in_specs=[_chunk_spec] * 8 + [_dinv_spec, _b_spec, _w_spec],
        out_specs=[_chunk_spec] * 4, out_shape=_chunk_out,
    )(*sc, *gc, dinv, b, w)


def _last_layer(sc, gc, dinv, b):
    return pl.pallas_call(
        _mm_last, grid=_grid,
        in_specs=[_chunk_spec] * 8 + [_dinv_spec, _b_spec],
        out_specs=_row_spec, out_shape=_row_out,
    )(*sc, *gc, dinv, b)


_NBUF = 4


def _sc_body(zeros, srcr, dstr, g0, g1, g2, g3, o0, o1, o2, o3,
             src_idx, dst_idx, rows, gsems, ssems, acc):
    c = lax.axis_index("c")
    s = lax.axis_index("s")
    row0 = s * _RPS

    def run_chunk(g_ref, out_ref):
        pltpu.sync_copy(zeros.at[pl.ds(row0, _RPS)], acc.at[pl.ds(row0, _RPS)])
        plsc.subcore_barrier()
        ebase = s * _ERPS

        def outer(i, carry):
            base = ebase + i * _NB
            pltpu.sync_copy(srcr.at[pl.ds(base, _NB)], src_idx)
            pltpu.sync_copy(dstr.at[pl.ds(base, _NB)], dst_idx)
            gcp = [None] * _NB
            scp = [None] * _NB

            def gather(j):
                b = j % _NBUF
                gcp[j] = pltpu.async_copy(
                    g_ref.at[src_idx.at[j]], rows.at[b], gsems.at[b])

            for j in range(_NBUF - 1):
                gather(j)
            for j in range(_NB):
                b = j % _NBUF
                gcp[j].wait()
                scp[j] = pltpu.async_copy(
                    rows.at[b], acc.at[dst_idx.at[j]], ssems.at[b], add=True)
                nxt = j + _NBUF - 1
                if nxt < _NB:
                    if nxt >= _NBUF:
                        scp[nxt - _NBUF].wait()
                    gather(nxt)
            for j in range(_NB - _NBUF, _NB):
                scp[j].wait()
            return carry

        lax.fori_loop(0, _NOUT, outer, 0)
        plsc.subcore_barrier()
        pltpu.sync_copy(acc.at[pl.ds(row0, _RPS)], out_ref.at[pl.ds(row0, _RPS)])

    @pl.when(c == 0)
    def _():
        run_chunk(g0, o0)
        run_chunk(g1, o1)

    @pl.when(c == 1)
    def _():
        run_chunk(g2, o2)
        run_chunk(g3, o3)


_sc_scatter = pl.kernel(
    _sc_body,
    out_type=[jax.ShapeDtypeStruct((_NPAD, _C), jnp.float32)] * 4,
    mesh=plsc.VectorSubcoreMesh(core_axis_name="c", subcore_axis_name="s"),
    compiler_params=pltpu.CompilerParams(use_tc_tiling_on_sc=False),
    scratch_types=[
        pltpu.VMEM((_NB, 128), jnp.int32),
        pltpu.VMEM((_NB, 128), jnp.int32),
        pltpu.VMEM((_NBUF, 128, _C), jnp.float32),
        pltpu.SemaphoreType.DMA((_NBUF,)),
        pltpu.SemaphoreType.DMA((_NBUF,)),
        pltpu.VMEM_SHARED((_NPAD, _C), jnp.float32),
    ],
)


def kernel(x, edge_index, batch, W1, b1, W2, b2, W3, b3):
    src = edge_index[0]
    dst = edge_index[1]
    indeg = jax.ops.segment_sum(jnp.ones((_E,), jnp.float32), dst,
                                num_segments=_N)
    dinv = jax.lax.rsqrt(indeg + 1.0)
    dinv = jnp.pad(dinv, (0, _NPAD - _N)).reshape(_NPAD, 1)

    x_pad = jnp.pad(x, ((0, _NPAD - _N), (0, _H - x.shape[1])))
    w1_pad = jnp.pad(W1, ((0, _H - W1.shape[0]), (0, 0)))

    pad_i = jnp.full((_EPAD - _E,), _NPAD - 1, jnp.int32)
    srcp = jnp.concatenate([src, pad_i]).reshape(_EROWS, 128)
    dstp = jnp.concatenate([dst, pad_i]).reshape(_EROWS, 128)
    zeros = jnp.zeros((_NPAD, _C), jnp.float32)

    def scat(gc):
        return _sc_scatter(zeros, srcp, dstp, *gc)

    g1 = _first_layer(x_pad, w1_pad, dinv)
    s1 = scat(g1)
    g2 = _mid_layer(s1, g1, dinv, b1.reshape(1, _H), W2)
    s2 = scat(g2)
    g3 = _mid_layer(s2, g2, dinv, b2.reshape(1, _H), W3)
    s3 = scat(g3)
    h3 = _last_layer(s3, g3, dinv, b3.reshape(1, _H))
    return jax.ops.segment_max(h3[:_N], batch, num_segments=_G)
